# Initial kernel scaffold; baseline (speedup 1.0000x reference)
import jax
import jax.numpy as jnp
from jax.experimental import pallas as pl


def kernel(w1, b1, w2, b2, wfc, bfc, x_nchw):
    raise NotImplementedError("write your pallas kernel here")


# speedup vs baseline: 14.5411x; 14.5411x over previous
import jax
import jax.numpy as jnp
from jax import lax
from jax.experimental import pallas as pl
from jax.experimental.pallas import tpu as pltpu

BB = 64


def _cparams():
    return pltpu.CompilerParams(
        dimension_semantics=("parallel",),
        vmem_limit_bytes=64 * 1024 * 1024,
    )


def _enc_kernel(x_ref, w1_ref, b1_ref, w2_ref, b2_ref, wfc_ref, bfc_ref,
                o_ref, s):
    B = x_ref.shape[0]
    f32 = jnp.float32

    def dot(a, w):
        return jnp.dot(a, w, preferred_element_type=f32)

    xc = x_ref[...].reshape(B * 8, 256)
    ra_e = dot(xc, w1_ref[0])
    rb_e = dot(xc, w1_ref[1])
    rc_e = dot(xc, w1_ref[2])
    rd_e = dot(xc, w1_ref[3])
    ra_o = dot(xc, w1_ref[4])
    rb_o = dot(xc, w1_ref[5])
    rc_o = dot(xc, w1_ref[6])
    rd_o = dot(xc, w1_ref[7])
    b1 = b1_ref[...]
    pe = jnp.maximum(
        jnp.maximum(jnp.maximum(ra_e, rb_e), jnp.maximum(ra_o, rb_o)) + b1,
        0.0).reshape(B, 8, 448)
    po = jnp.maximum(
        jnp.maximum(jnp.maximum(rc_e, rd_e), jnp.maximum(rc_o, rd_o)) + b1,
        0.0).reshape(B, 8, 448)

    z1 = jnp.zeros((B, 1, 448), f32)
    s[:, 0:1, 0:448] = z1
    s[:, 6:7, 1344:1792] = z1
    s[:, 0:7, 448:896] = pe[:, 0:7]
    s[:, 0:6, 1344:1792] = pe[:, 1:7]
    s[:, 1:8, 0:448] = po[:, 0:7]
    s[:, 0:7, 896:1344] = po[:, 0:7]

    sf = s[...].reshape(B * 8, 1792)
    fa_e = dot(sf, w2_ref[0])
    fb_e = dot(sf, w2_ref[1])
    fa_o = dot(sf, w2_ref[2])
    fb_o = dot(sf, w2_ref[3])
    b2 = b2_ref[...]
    p2 = jnp.maximum(
        jnp.maximum(jnp.maximum(fa_e, fb_e), jnp.maximum(fa_o, fb_o)) + b2,
        0.0).reshape(B, 8, 448)

    acc = jnp.zeros((B, 10), f32)
    for k in range(7):
        acc = acc + dot(p2[:, k, :], wfc_ref[k])
    y = acc + bfc_ref[...]
    ss = jnp.sum(y * y, axis=-1, keepdims=True)
    o_ref[...] = y * lax.rsqrt(jnp.maximum(ss, 1e-24))


@jax.jit
def _forward(w1, b1, w2, b2, wfc, bfc, x_nchw):
    n = x_nchw.shape[0]
    f32 = jnp.float32

    w1r = w1.reshape(3, 3, 32)
    w1e = jnp.zeros((3, 32, 14, 32), f32)
    w1o = jnp.zeros((3, 32, 14, 32), f32)
    for u in range(14):
        w1e = w1e.at[:, 2 * u:2 * u + 3, u, :].set(w1r)
        w1o = w1o.at[:, 2 * u + 1:2 * u + 4, u, :].set(w1r)
    w1e = w1e.reshape(3, 32, 448)
    w1o = w1o.reshape(3, 32, 448)
    z32 = jnp.zeros((32, 448), f32)

    def stk1(ws, lead):
        blocks = [z32] * lead + list(ws) + [z32] * (8 - 3 - lead)
        return jnp.concatenate(blocks, axis=0)

    w1s = jnp.stack([stk1(w1e, 0), stk1(w1e, 1), stk1(w1e, 2), stk1(w1e, 3),
                     stk1(w1o, 0), stk1(w1o, 1), stk1(w1o, 2), stk1(w1o, 3)])
    b1t = jnp.tile(b1.reshape(1, 1, 32), (1, 14, 1)).reshape(1, 448)

    w2r = w2.reshape(3, 3, 32, 64)
    w2ep = jnp.zeros((3, 16, 32, 7, 64), f32)
    w2op = jnp.zeros((3, 16, 32, 7, 64), f32)
    for v in range(7):
        w2ep = w2ep.at[:, 2 * v:2 * v + 3, :, v, :].set(w2r)
        w2op = w2op.at[:, 2 * v + 1:2 * v + 4, :, v, :].set(w2r)
    w2e = w2ep[:, 1:15].reshape(3, 448, 448)
    w2o = w2op[:, 1:15].reshape(3, 448, 448)
    z448 = jnp.zeros((448, 448), f32)
    w2s = jnp.stack([
        jnp.concatenate([w2e[0], w2e[1], w2e[2], z448], axis=0),
        jnp.concatenate([z448, w2e[0], w2e[1], w2e[2]], axis=0),
        jnp.concatenate([w2o[0], w2o[1], w2o[2], z448], axis=0),
        jnp.concatenate([z448, w2o[0], w2o[1], w2o[2]], axis=0),
    ])
    b2t = jnp.tile(b2.reshape(1, 1, 64), (1, 7, 1)).reshape(1, 448)

    wfc7 = wfc.reshape(7, 448, 10)
    bfcr = bfc.reshape(1, 10)

    x = x_nchw.reshape(n, 28, 28)
    xpad = jnp.pad(x, ((0, 0), (1, 3), (1, 3)))
    ph = xpad.reshape(n, 8, 4, 32)
    zrow = jnp.zeros((n, 1, 4, 32), f32)
    ph1 = jnp.concatenate([ph[:, 1:8], zrow], axis=1)
    xc2 = jnp.concatenate([ph, ph1], axis=2).reshape(n, 8, 256)

    out = pl.pallas_call(
        _enc_kernel,
        grid=(n // BB,),
        in_specs=[
            pl.BlockSpec((BB, 8, 256), lambda i: (i, 0, 0)),
            pl.BlockSpec((8, 256, 448), lambda i: (0, 0, 0)),
            pl.BlockSpec((1, 448), lambda i: (0, 0)),
            pl.BlockSpec((4, 1792, 448), lambda i: (0, 0, 0)),
            pl.BlockSpec((1, 448), lambda i: (0, 0)),
            pl.BlockSpec((7, 448, 10), lambda i: (0, 0, 0)),
            pl.BlockSpec((1, 10), lambda i: (0, 0)),
        ],
        out_specs=pl.BlockSpec((BB, 10), lambda i: (i, 0)),
        out_shape=jax.ShapeDtypeStruct((n, 10), f32),
        scratch_shapes=[pltpu.VMEM((BB, 8, 1792), f32)],
        compiler_params=_cparams(),
        cost_estimate=pl.CostEstimate(
            flops=2 * n * (36 * 28 * 7 * 32 + 18 * 112 * 32 * 64 + 3136 * 10),
            transcendentals=n,
            bytes_accessed=4 * n * (8 * 256 + 10)),
    )(xc2, w1s, b1t, w2s, b2t, wfc7, bfcr)
    return out


def kernel(w1, b1, w2, b2, wfc, bfc, x_nchw):
    return _forward(w1, b1, w2, b2, wfc, bfc, x_nchw)


# speedup vs baseline: 15.7428x; 1.0826x over previous
import jax
import jax.numpy as jnp
from jax import lax
from jax.experimental import pallas as pl
from jax.experimental.pallas import tpu as pltpu

BB = 128


def _cparams():
    return pltpu.CompilerParams(
        dimension_semantics=("parallel",),
        vmem_limit_bytes=64 * 1024 * 1024,
    )


def _enc_kernel(x_ref, w1_ref, b1_ref, w2_ref, b2_ref, wfc_ref, bfc_ref,
                o_ref, s):
    B = x_ref.shape[0]
    f32 = jnp.float32

    def dot(a, w):
        return jnp.dot(a, w, preferred_element_type=f32)

    bf16 = jnp.bfloat16
    xc = x_ref[...].reshape(B * 8, 256)
    ra_e = dot(xc, w1_ref[0])
    rb_e = dot(xc, w1_ref[1])
    rc_e = dot(xc, w1_ref[2])
    rd_e = dot(xc, w1_ref[3])
    ra_o = dot(xc, w1_ref[4])
    rb_o = dot(xc, w1_ref[5])
    rc_o = dot(xc, w1_ref[6])
    rd_o = dot(xc, w1_ref[7])
    b1 = b1_ref[...]
    pe = jnp.maximum(
        jnp.maximum(jnp.maximum(ra_e, rb_e), jnp.maximum(ra_o, rb_o)) + b1,
        0.0).reshape(B, 8, 448)
    po = jnp.maximum(
        jnp.maximum(jnp.maximum(rc_e, rd_e), jnp.maximum(rc_o, rd_o)) + b1,
        0.0).reshape(B, 8, 448)

    peb = pe.astype(bf16)
    pob = po.astype(bf16)
    z1 = jnp.zeros((B, 1, 448), bf16)
    s[:, 0:1, 0:448] = z1
    s[:, 6:7, 1344:1792] = z1
    s[:, 0:7, 448:896] = peb[:, 0:7]
    s[:, 0:6, 1344:1792] = peb[:, 1:7]
    s[:, 1:8, 0:448] = pob[:, 0:7]
    s[:, 0:7, 896:1344] = pob[:, 0:7]

    sf = s[...].reshape(B * 8, 1792)
    fa_e = dot(sf, w2_ref[0])
    fb_e = dot(sf, w2_ref[1])
    fa_o = dot(sf, w2_ref[2])
    fb_o = dot(sf, w2_ref[3])
    b2 = b2_ref[...]
    p2 = jnp.maximum(
        jnp.maximum(jnp.maximum(fa_e, fb_e), jnp.maximum(fa_o, fb_o)) + b2,
        0.0).reshape(B, 8, 448).astype(bf16)

    acc = jnp.zeros((B, 10), f32)
    for k in range(7):
        acc = acc + dot(p2[:, k, :], wfc_ref[k])
    y = acc + bfc_ref[...]
    ss = jnp.sum(y * y, axis=-1, keepdims=True)
    o_ref[...] = y * lax.rsqrt(jnp.maximum(ss, 1e-24))


@jax.jit
def _forward(w1, b1, w2, b2, wfc, bfc, x_nchw):
    n = x_nchw.shape[0]
    f32 = jnp.float32

    w1r = w1.reshape(3, 3, 32)
    w1e = jnp.zeros((3, 32, 14, 32), f32)
    w1o = jnp.zeros((3, 32, 14, 32), f32)
    for u in range(14):
        w1e = w1e.at[:, 2 * u:2 * u + 3, u, :].set(w1r)
        w1o = w1o.at[:, 2 * u + 1:2 * u + 4, u, :].set(w1r)
    w1e = w1e.reshape(3, 32, 448)
    w1o = w1o.reshape(3, 32, 448)
    z32 = jnp.zeros((32, 448), f32)

    def stk1(ws, lead):
        blocks = [z32] * lead + list(ws) + [z32] * (8 - 3 - lead)
        return jnp.concatenate(blocks, axis=0)

    w1s = jnp.stack([stk1(w1e, 0), stk1(w1e, 1), stk1(w1e, 2), stk1(w1e, 3),
                     stk1(w1o, 0), stk1(w1o, 1), stk1(w1o, 2), stk1(w1o, 3)])
    b1t = jnp.tile(b1.reshape(1, 1, 32), (1, 14, 1)).reshape(1, 448)

    w2r = w2.reshape(3, 3, 32, 64)
    w2ep = jnp.zeros((3, 16, 32, 7, 64), f32)
    w2op = jnp.zeros((3, 16, 32, 7, 64), f32)
    for v in range(7):
        w2ep = w2ep.at[:, 2 * v:2 * v + 3, :, v, :].set(w2r)
        w2op = w2op.at[:, 2 * v + 1:2 * v + 4, :, v, :].set(w2r)
    w2e = w2ep[:, 1:15].reshape(3, 448, 448)
    w2o = w2op[:, 1:15].reshape(3, 448, 448)
    z448 = jnp.zeros((448, 448), f32)
    w2s = jnp.stack([
        jnp.concatenate([w2e[0], w2e[1], w2e[2], z448], axis=0),
        jnp.concatenate([z448, w2e[0], w2e[1], w2e[2]], axis=0),
        jnp.concatenate([w2o[0], w2o[1], w2o[2], z448], axis=0),
        jnp.concatenate([z448, w2o[0], w2o[1], w2o[2]], axis=0),
    ])
    b2t = jnp.tile(b2.reshape(1, 1, 64), (1, 7, 1)).reshape(1, 448)

    bf16 = jnp.bfloat16
    wfc7 = wfc.reshape(7, 448, 10).astype(bf16)
    bfcr = bfc.reshape(1, 10)
    w1s = w1s.astype(bf16)
    w2s = w2s.astype(bf16)

    x = x_nchw.reshape(n, 28, 28)
    xpad = jnp.pad(x, ((0, 0), (1, 3), (1, 3)))
    ph = xpad.reshape(n, 8, 4, 32)
    zrow = jnp.zeros((n, 1, 4, 32), f32)
    ph1 = jnp.concatenate([ph[:, 1:8], zrow], axis=1)
    xc2 = jnp.concatenate([ph, ph1], axis=2).reshape(n, 8, 256).astype(bf16)

    out = pl.pallas_call(
        _enc_kernel,
        grid=(n // BB,),
        in_specs=[
            pl.BlockSpec((BB, 8, 256), lambda i: (i, 0, 0)),
            pl.BlockSpec((8, 256, 448), lambda i: (0, 0, 0)),
            pl.BlockSpec((1, 448), lambda i: (0, 0)),
            pl.BlockSpec((4, 1792, 448), lambda i: (0, 0, 0)),
            pl.BlockSpec((1, 448), lambda i: (0, 0)),
            pl.BlockSpec((7, 448, 10), lambda i: (0, 0, 0)),
            pl.BlockSpec((1, 10), lambda i: (0, 0)),
        ],
        out_specs=pl.BlockSpec((BB, 10), lambda i: (i, 0)),
        out_shape=jax.ShapeDtypeStruct((n, 10), f32),
        scratch_shapes=[pltpu.VMEM((BB, 8, 1792), jnp.bfloat16)],
        compiler_params=_cparams(),
        cost_estimate=pl.CostEstimate(
            flops=2 * n * (36 * 28 * 7 * 32 + 18 * 112 * 32 * 64 + 3136 * 10),
            transcendentals=n,
            bytes_accessed=4 * n * (8 * 256 + 10)),
    )(xc2, w1s, b1t, w2s, b2t, wfc7, bfcr)
    return out


def kernel(w1, b1, w2, b2, wfc, bfc, x_nchw):
    return _forward(w1, b1, w2, b2, wfc, bfc, x_nchw)


# speedup vs baseline: 16.2280x; 1.0308x over previous
import jax
import jax.numpy as jnp
from jax import lax
from jax.experimental import pallas as pl
from jax.experimental.pallas import tpu as pltpu

BB = 128


def _cparams():
    return pltpu.CompilerParams(
        dimension_semantics=("parallel",),
        vmem_limit_bytes=64 * 1024 * 1024,
    )


def _enc_kernel(x_ref, w1_ref, b1_ref, w2_ref, b2_ref, wfc_ref, bfc_ref,
                o_ref, s):
    B = x_ref.shape[0]
    f32 = jnp.float32

    def dot(a, w):
        return jnp.dot(a, w, preferred_element_type=f32)

    bf16 = jnp.bfloat16
    xq = x_ref[...]
    xc = jnp.concatenate(
        [xq[:, 0:8, :].reshape(B * 8, 128),
         xq[:, 1:9, :].reshape(B * 8, 128)],
        axis=1)
    ra_e = dot(xc, w1_ref[0])
    rb_e = dot(xc, w1_ref[1])
    rc_e = dot(xc, w1_ref[2])
    rd_e = dot(xc, w1_ref[3])
    ra_o = dot(xc, w1_ref[4])
    rb_o = dot(xc, w1_ref[5])
    rc_o = dot(xc, w1_ref[6])
    rd_o = dot(xc, w1_ref[7])
    b1 = b1_ref[...]
    pe = jnp.maximum(
        jnp.maximum(jnp.maximum(ra_e, rb_e), jnp.maximum(ra_o, rb_o)) + b1,
        0.0).reshape(B, 8, 448)
    po = jnp.maximum(
        jnp.maximum(jnp.maximum(rc_e, rd_e), jnp.maximum(rc_o, rd_o)) + b1,
        0.0).reshape(B, 8, 448)

    peb = pe.astype(bf16)
    pob = po.astype(bf16)
    z1 = jnp.zeros((B, 1, 448), bf16)
    s[:, 0:1, 0:448] = z1
    s[:, 6:7, 1344:1792] = z1
    s[:, 0:7, 448:896] = peb[:, 0:7]
    s[:, 0:6, 1344:1792] = peb[:, 1:7]
    s[:, 1:8, 0:448] = pob[:, 0:7]
    s[:, 0:7, 896:1344] = pob[:, 0:7]

    sf = s[...].reshape(B * 8, 1792)
    fa_e = dot(sf, w2_ref[0])
    fb_e = dot(sf, w2_ref[1])
    fa_o = dot(sf, w2_ref[2])
    fb_o = dot(sf, w2_ref[3])
    b2 = b2_ref[...]
    p2 = jnp.maximum(
        jnp.maximum(jnp.maximum(fa_e, fb_e), jnp.maximum(fa_o, fb_o)) + b2,
        0.0).reshape(B, 8, 448).astype(bf16)

    acc = jnp.zeros((B, 10), f32)
    for k in range(7):
        acc = acc + dot(p2[:, k, :], wfc_ref[k])
    y = acc + bfc_ref[...]
    ss = jnp.sum(y * y, axis=-1, keepdims=True)
    o_ref[...] = y * lax.rsqrt(jnp.maximum(ss, 1e-24))


@jax.jit
def _forward(w1, b1, w2, b2, wfc, bfc, x_nchw):
    n = x_nchw.shape[0]
    f32 = jnp.float32

    w1r = w1.reshape(3, 3, 32)
    w1e = jnp.zeros((3, 32, 14, 32), f32)
    w1o = jnp.zeros((3, 32, 14, 32), f32)
    for u in range(14):
        w1e = w1e.at[:, 2 * u:2 * u + 3, u, :].set(w1r)
        w1o = w1o.at[:, 2 * u + 1:2 * u + 4, u, :].set(w1r)
    w1e = w1e.reshape(3, 32, 448)
    w1o = w1o.reshape(3, 32, 448)
    z32 = jnp.zeros((32, 448), f32)

    def stk1(ws, lead):
        blocks = [z32] * lead + list(ws) + [z32] * (8 - 3 - lead)
        return jnp.concatenate(blocks, axis=0)

    w1s = jnp.stack([stk1(w1e, 0), stk1(w1e, 1), stk1(w1e, 2), stk1(w1e, 3),
                     stk1(w1o, 0), stk1(w1o, 1), stk1(w1o, 2), stk1(w1o, 3)])
    b1t = jnp.tile(b1.reshape(1, 1, 32), (1, 14, 1)).reshape(1, 448)

    w2r = w2.reshape(3, 3, 32, 64)
    w2ep = jnp.zeros((3, 16, 32, 7, 64), f32)
    w2op = jnp.zeros((3, 16, 32, 7, 64), f32)
    for v in range(7):
        w2ep = w2ep.at[:, 2 * v:2 * v + 3, :, v, :].set(w2r)
        w2op = w2op.at[:, 2 * v + 1:2 * v + 4, :, v, :].set(w2r)
    w2e = w2ep[:, 1:15].reshape(3, 448, 448)
    w2o = w2op[:, 1:15].reshape(3, 448, 448)
    z448 = jnp.zeros((448, 448), f32)
    w2s = jnp.stack([
        jnp.concatenate([w2e[0], w2e[1], w2e[2], z448], axis=0),
        jnp.concatenate([z448, w2e[0], w2e[1], w2e[2]], axis=0),
        jnp.concatenate([w2o[0], w2o[1], w2o[2], z448], axis=0),
        jnp.concatenate([z448, w2o[0], w2o[1], w2o[2]], axis=0),
    ])
    b2t = jnp.tile(b2.reshape(1, 1, 64), (1, 7, 1)).reshape(1, 448)

    bf16 = jnp.bfloat16
    wfc7 = wfc.reshape(7, 448, 10).astype(bf16)
    bfcr = bfc.reshape(1, 10)
    w1s = w1s.astype(bf16)
    w2s = w2s.astype(bf16)

    x = x_nchw.reshape(n, 28, 28)
    xpad = jnp.pad(x, ((0, 0), (1, 7), (1, 3)))
    xc2 = xpad.reshape(n, 9, 128).astype(bf16)

    out = pl.pallas_call(
        _enc_kernel,
        grid=(n // BB,),
        in_specs=[
            pl.BlockSpec((BB, 9, 128), lambda i: (i, 0, 0)),
            pl.BlockSpec((8, 256, 448), lambda i: (0, 0, 0)),
            pl.BlockSpec((1, 448), lambda i: (0, 0)),
            pl.BlockSpec((4, 1792, 448), lambda i: (0, 0, 0)),
            pl.BlockSpec((1, 448), lambda i: (0, 0)),
            pl.BlockSpec((7, 448, 10), lambda i: (0, 0, 0)),
            pl.BlockSpec((1, 10), lambda i: (0, 0)),
        ],
        out_specs=pl.BlockSpec((BB, 10), lambda i: (i, 0)),
        out_shape=jax.ShapeDtypeStruct((n, 10), f32),
        scratch_shapes=[pltpu.VMEM((BB, 8, 1792), jnp.bfloat16)],
        compiler_params=_cparams(),
        cost_estimate=pl.CostEstimate(
            flops=2 * n * (36 * 28 * 7 * 32 + 18 * 112 * 32 * 64 + 3136 * 10),
            transcendentals=n,
            bytes_accessed=n * (2 * 9 * 128 + 4 * 10)),
    )(xc2, w1s, b1t, w2s, b2t, wfc7, bfcr)
    return out


def kernel(w1, b1, w2, b2, wfc, bfc, x_nchw):
    return _forward(w1, b1, w2, b2, wfc, bfc, x_nchw)
